# baseline (speedup 1.0000x reference)
import functools

import jax
import jax.numpy as jnp
from jax.experimental import pallas as pl
from jax.experimental.pallas import tpu as pltpu

_EPS = 1e-5


def _gn_act(u, w, b, is_final, feat):
    g = w * (u * jax.lax.rsqrt(u * u + _EPS)) + b
    if is_final:
        return jnp.maximum(feat + g, 0.0)
    return jnp.maximum(g, 0.1 * g)


def _fused_kernel(nb, B, n, d, x_ref, A_ref, cb_ref, gnw_ref, gnb_ref,
                  lw_ref, lb_ref, out_ref, halo_ref):
    j = pl.program_id(0)
    P = B // 2

    @pl.when(j == 0)
    def _init():
        halo_ref[...] = jnp.zeros_like(halo_ref)
        out_ref[...] = jnp.zeros_like(out_ref)

    feat = x_ref[...].reshape(P, 2 * d)

    x = feat
    for i in range(4):
        h = halo_ref[i, 0:2, :]
        halo_ref[i, 0:2, :] = x[P - 2:P, :]
        ext = jnp.concatenate([h, x], axis=0)
        zcat = jnp.concatenate(
            [x, ext[1:P + 1], ext[0:P]], axis=1)
        u = jax.lax.dot_general(
            zcat, A_ref[i], (((1,), (0,)), ((), ())),
            preferred_element_type=jnp.float32) + cb_ref[i][None, :]
        x = _gn_act(u, gnw_ref[i][None, :], gnb_ref[i][None, :], i == 3, feat)

    out_ref[...] += jnp.sum(x, axis=0, keepdims=True)

    @pl.when(j == nb - 1)
    def _finish():
        fl = feat[P - 1:P, d:2 * d]
        v = fl
        for i in range(4):
            u = jax.lax.dot_general(
                v, A_ref[i][0:d, 0:d], (((1,), (0,)), ((), ())),
                preferred_element_type=jnp.float32) + cb_ref[i][None, 0:d]
            v = _gn_act(u, gnw_ref[i][None, 0:d], gnb_ref[i][None, 0:d],
                        i == 3, fl)
        acc = out_ref[...] + jnp.concatenate(
            [jnp.zeros((1, d), jnp.float32), v - x[P - 1:P, d:2 * d]], axis=1)
        pooled = (acc[:, 0:d] + acc[:, d:2 * d]) * (1.0 / n)
        t = jax.lax.dot_general(
            pooled, lw_ref[...], (((1,), (1,)), ((), ())),
            preferred_element_type=jnp.float32,
            precision=jax.lax.Precision.HIGHEST) + lb_ref[...][None, :]
        res = jnp.maximum(t, 0.0) + jnp.log1p(jnp.exp(-jnp.abs(t)))
        out_ref[...] = jnp.concatenate([res, jnp.zeros_like(res)], axis=1)


def _pick_block(n):
    for cand in (20000, 10000, 5000, 4000, 2000, 1000, 500, 200, 100, 40, 16, 8):
        if n % cand == 0 and cand % 4 == 0:
            return cand
    return n


@jax.jit
def kernel(feat, conv_w, conv_b, gn_w, gn_b, gn_ms, lin_w, lin_b):
    n, d = feat.shape[1], feat.shape[2]
    x = feat.reshape(n, d)
    c = (1.0 - gn_ms)[:, None, :]
    A0 = (conv_w[:, 0] - conv_w[:, 2]) * c
    A1 = (3.0 * conv_w[:, 3] - conv_w[:, 1]) * c
    A2 = 2.0 * conv_w[:, 2] * c
    A3 = -4.0 * conv_w[:, 3] * c
    Z = jnp.zeros_like(A0)
    row = lambda l, r: jnp.concatenate([l, r], axis=2)
    A = jnp.concatenate(
        [row(A0, A1), row(Z, A0),
         row(A2, A3), row(A1, A2),
         row(Z, Z), row(A3, Z)],
        axis=1)
    t2 = lambda p: jnp.concatenate([p, p], axis=1)
    cb = t2(conv_b * (1.0 - gn_ms))
    gnw2, gnb2 = t2(gn_w), t2(gn_b)

    B = _pick_block(n)
    nb = n // B
    full = lambda s: pl.BlockSpec(s, lambda j: (0,) * len(s))
    out = pl.pallas_call(
        functools.partial(_fused_kernel, nb, B, n, d),
        grid=(nb,),
        in_specs=[
            pl.BlockSpec((B, d), lambda j: (j, 0)),
            full((4, 6 * d, 2 * d)),
            full((4, 2 * d)),
            full((4, 2 * d)),
            full((4, 2 * d)),
            full((d, d)),
            full((d,)),
        ],
        out_specs=pl.BlockSpec((1, 2 * d), lambda j: (0, 0)),
        out_shape=jax.ShapeDtypeStruct((1, 2 * d), jnp.float32),
        scratch_shapes=[pltpu.VMEM((4, 8, 2 * d), jnp.float32)],
    )(x, A, cb, gnw2, gnb2, lin_w, lin_b)
    return out[0, 0:d]
